# baseline (speedup 1.0000x reference)
import functools
import jax
import jax.numpy as jnp
from jax import lax
from jax.experimental import pallas as pl
from jax.experimental.pallas import tpu as pltpu
from jax.experimental.pallas import tpu_sc as plsc

NC = 2
NS = 16
NW = NC * NS
LANES = 16
GROUP = 128
GPB = 4
BATCH = GROUP * GPB
DEPTH = 3


def _make_sc_kernel(n_nodes: int, d: int, e_edges: int):
    assert d == 32
    assert e_edges % GROUP == 0
    nblk = e_edges // GROUP
    blk_lo = nblk // NW
    extra = nblk - blk_lo * NW
    n_batches = blk_lo // GPB
    tail_lo = blk_lo - n_batches * GPB
    assert n_batches >= 6 and n_batches % 2 == 1

    mesh = plsc.VectorSubcoreMesh(
        core_axis_name="c", subcore_axis_name="s",
        num_cores=NC, num_subcores=NS)

    @functools.partial(
        pl.kernel,
        out_type=jax.ShapeDtypeStruct((e_edges,), jnp.float32),
        mesh=mesh,
        compiler_params=pltpu.CompilerParams(
            needs_layout_passes=False, use_tc_tiling_on_sc=False),
        scratch_types=[
            pltpu.VMEM((DEPTH * GPB, GROUP), jnp.int32),
            pltpu.VMEM((DEPTH * GPB, GROUP), jnp.int32),
            pltpu.VMEM((DEPTH, BATCH, 32), jnp.float32),
            pltpu.VMEM((DEPTH, BATCH, 32), jnp.float32),
            pltpu.VMEM((DEPTH, BATCH), jnp.float32),
            [pltpu.SemaphoreType.DMA] * DEPTH,
            [pltpu.SemaphoreType.DMA] * DEPTH,
            [pltpu.SemaphoreType.DMA] * DEPTH,
        ],
    )
    def k(z_hbm, ei_hbm, out_hbm, sidx, didx, srow, drow, obuf,
          gsems, isems, osems):
        wid = lax.axis_index("s") * NC + lax.axis_index("c")
        base_blk = wid * blk_lo + jnp.minimum(wid, extra)
        n_my_extra = jnp.where(wid < extra, 1, 0)
        lane = lax.iota(jnp.int32, LANES)

        def fire_idx(b, r):
            for j in range(GPB):
                blk = base_blk + b * GPB + j
                pltpu.async_copy(ei_hbm.at[blk, 0, :],
                                 sidx.at[r * GPB + j], isems[r])
                pltpu.async_copy(ei_hbm.at[blk, 1, :],
                                 didx.at[r * GPB + j], isems[r])

        def drain_idx(r):
            for j in range(GPB):
                pltpu.make_async_copy(ei_hbm.at[0, 0, :],
                                      sidx.at[r * GPB + j], isems[r]).wait()
                pltpu.make_async_copy(ei_hbm.at[0, 1, :],
                                      didx.at[r * GPB + j], isems[r]).wait()

        def fire_gathers(r):
            for j in range(GPB):
                pltpu.async_copy(z_hbm.at[sidx.at[r * GPB + j]],
                                 srow.at[r, pl.ds(j * GROUP, GROUP), :],
                                 gsems[r])
                pltpu.async_copy(z_hbm.at[didx.at[r * GPB + j]],
                                 drow.at[r, pl.ds(j * GROUP, GROUP), :],
                                 gsems[r])

        def drain_gathers(r):
            for j in range(GPB):
                pltpu.make_async_copy(z_hbm.at[sidx.at[r * GPB + j]],
                                      srow.at[r, pl.ds(j * GROUP, GROUP), :],
                                      gsems[r]).wait()
                pltpu.make_async_copy(z_hbm.at[didx.at[r * GPB + j]],
                                      drow.at[r, pl.ds(j * GROUP, GROUP), :],
                                      gsems[r]).wait()

        def dot_groups(r, n_groups):
            def group_body(g, c2):
                acc = jnp.zeros((LANES,), jnp.float32)
                for e in range(LANES):
                    q = g * LANES + e
                    s0 = (srow[r, q, pl.ds(0, 16)] * drow[r, q, pl.ds(0, 16)]
                          + srow[r, q, pl.ds(16, 16)] * drow[r, q, pl.ds(16, 16)])
                    acc = jnp.where(lane == e, jnp.sum(s0), acc)
                obuf[r, pl.ds(g * LANES, LANES)] = 1.0 / (1.0 + jnp.exp(-acc))
                return c2
            lax.fori_loop(0, n_groups, group_body, 0, unroll=False)

        def fire_out(b, r):
            for j in range(GPB):
                blk = base_blk + b * GPB + j
                pltpu.async_copy(obuf.at[r, pl.ds(j * GROUP, GROUP)],
                                 out_hbm.at[pl.ds(blk * GROUP, GROUP)],
                                 osems[r])

        def drain_out(r):
            for j in range(GPB):
                pltpu.make_async_copy(obuf.at[r, pl.ds(j * GROUP, GROUP)],
                                      out_hbm.at[pl.ds(0, GROUP)],
                                      osems[r]).wait()

        def compute(b, r):
            @pl.when(b >= DEPTH)
            def _():
                drain_out(r)
            dot_groups(r, BATCH // LANES)
            fire_out(b, r)

        def one_block(blk, valid):
            @pl.when(valid)
            def _():
                pltpu.async_copy(ei_hbm.at[blk, 0, :], sidx.at[0], isems[0])
                pltpu.async_copy(ei_hbm.at[blk, 1, :], didx.at[0], isems[0])
                pltpu.make_async_copy(ei_hbm.at[0, 0, :], sidx.at[0],
                                      isems[0]).wait()
                pltpu.make_async_copy(ei_hbm.at[0, 1, :], didx.at[0],
                                      isems[0]).wait()
                pltpu.async_copy(z_hbm.at[sidx.at[0]],
                                 srow.at[0, pl.ds(0, GROUP), :], gsems[0])
                pltpu.async_copy(z_hbm.at[didx.at[0]],
                                 drow.at[0, pl.ds(0, GROUP), :], gsems[0])
                pltpu.make_async_copy(z_hbm.at[sidx.at[0]],
                                      srow.at[0, pl.ds(0, GROUP), :],
                                      gsems[0]).wait()
                pltpu.make_async_copy(z_hbm.at[didx.at[0]],
                                      drow.at[0, pl.ds(0, GROUP), :],
                                      gsems[0]).wait()
                dot_groups(0, GROUP // LANES)
                pltpu.async_copy(obuf.at[0, pl.ds(0, GROUP)],
                                 out_hbm.at[pl.ds(blk * GROUP, GROUP)],
                                 osems[0])
                pltpu.make_async_copy(obuf.at[0, pl.ds(0, GROUP)],
                                      out_hbm.at[pl.ds(0, GROUP)],
                                      osems[0]).wait()

        for t in range(tail_lo):
            one_block(base_blk + n_batches * GPB + t, True)
        one_block(base_blk + n_batches * GPB + tail_lo, n_my_extra > 0)

        fire_idx(0, 0)
        fire_idx(1, 1)
        fire_idx(2, 2)
        drain_idx(0)
        fire_gathers(0)
        drain_idx(1)
        fire_gathers(1)

        def stepper(b, r, do_g, do_i):
            if do_g:
                drain_idx((r + 2) % DEPTH)
                fire_gathers((r + 2) % DEPTH)
            drain_gathers(r)
            if do_i:
                fire_idx(b + 3, r)
            compute(b, r)

        m3 = ((n_batches - 3) // 3) * 3

        def loop_body(i, carry):
            b0 = i * 3
            stepper(b0, 0, True, True)
            stepper(b0 + 1, 1, True, True)
            stepper(b0 + 2, 2, True, True)
            return carry

        lax.fori_loop(0, m3 // 3, loop_body, 0, unroll=False)
        for b in range(m3, n_batches):
            stepper(b, b % 3, b + 2 <= n_batches - 1, b + 3 <= n_batches - 1)
        for r in range(DEPTH):
            drain_out(r)

    return k


def kernel(z, edge_index):
    n_nodes, d = z.shape
    e = edge_index.shape[1]
    nblk = e // GROUP
    ei = edge_index.astype(jnp.int32)
    ei3 = ei.reshape(2, nblk, GROUP).transpose(1, 0, 2)
    return _make_sc_kernel(n_nodes, d, e)(z.astype(jnp.float32), ei3)


# speedup vs baseline: 1.0410x; 1.0410x over previous
import functools
import jax
import jax.numpy as jnp
from jax import lax
from jax.experimental import pallas as pl
from jax.experimental.pallas import tpu as pltpu
from jax.experimental.pallas import tpu_sc as plsc

NC = 2
NS = 16
NW = NC * NS
LANES = 16
GROUP = 128
GPB = 4
BATCH = GROUP * GPB
DEPTH = 3


def _make_sc_kernel(n_nodes: int, d: int, e_edges: int):
    assert d == 32
    assert e_edges % GROUP == 0
    nblk = e_edges // GROUP
    blk_lo = nblk // NW
    extra = nblk - blk_lo * NW
    n_batches = blk_lo // GPB
    tail_lo = blk_lo - n_batches * GPB
    assert n_batches >= 6 and n_batches % 2 == 1

    mesh = plsc.VectorSubcoreMesh(
        core_axis_name="c", subcore_axis_name="s",
        num_cores=NC, num_subcores=NS)

    @functools.partial(
        pl.kernel,
        out_type=jax.ShapeDtypeStruct((e_edges,), jnp.float32),
        mesh=mesh,
        compiler_params=pltpu.CompilerParams(
            needs_layout_passes=False, use_tc_tiling_on_sc=False),
        scratch_types=[
            pltpu.VMEM((DEPTH * GPB, GROUP), jnp.int32),
            pltpu.VMEM((DEPTH * GPB, GROUP), jnp.int32),
            pltpu.VMEM((DEPTH, BATCH, 32), jnp.bfloat16),
            pltpu.VMEM((DEPTH, BATCH, 32), jnp.bfloat16),
            pltpu.VMEM((DEPTH, BATCH), jnp.float32),
            [pltpu.SemaphoreType.DMA] * DEPTH,
            [pltpu.SemaphoreType.DMA] * DEPTH,
            [pltpu.SemaphoreType.DMA] * DEPTH,
        ],
    )
    def k(z_hbm, ei_hbm, out_hbm, sidx, didx, srow, drow, obuf,
          gsems, isems, osems):
        wid = lax.axis_index("s") * NC + lax.axis_index("c")
        base_blk = wid * blk_lo + jnp.minimum(wid, extra)
        n_my_extra = jnp.where(wid < extra, 1, 0)
        lane = lax.iota(jnp.int32, LANES)

        def fire_idx(b, r):
            for j in range(GPB):
                blk = base_blk + b * GPB + j
                pltpu.async_copy(ei_hbm.at[blk, 0, :],
                                 sidx.at[r * GPB + j], isems[r])
                pltpu.async_copy(ei_hbm.at[blk, 1, :],
                                 didx.at[r * GPB + j], isems[r])

        def drain_idx(r):
            for j in range(GPB):
                pltpu.make_async_copy(ei_hbm.at[0, 0, :],
                                      sidx.at[r * GPB + j], isems[r]).wait()
                pltpu.make_async_copy(ei_hbm.at[0, 1, :],
                                      didx.at[r * GPB + j], isems[r]).wait()

        def fire_gathers(r):
            for j in range(GPB):
                pltpu.async_copy(z_hbm.at[sidx.at[r * GPB + j]],
                                 srow.at[r, pl.ds(j * GROUP, GROUP), :],
                                 gsems[r])
                pltpu.async_copy(z_hbm.at[didx.at[r * GPB + j]],
                                 drow.at[r, pl.ds(j * GROUP, GROUP), :],
                                 gsems[r])

        def drain_gathers(r):
            for j in range(GPB):
                pltpu.make_async_copy(z_hbm.at[sidx.at[r * GPB + j]],
                                      srow.at[r, pl.ds(j * GROUP, GROUP), :],
                                      gsems[r]).wait()
                pltpu.make_async_copy(z_hbm.at[didx.at[r * GPB + j]],
                                      drow.at[r, pl.ds(j * GROUP, GROUP), :],
                                      gsems[r]).wait()

        def dot_groups(r, n_groups):
            def group_body(g, c2):
                acc = jnp.zeros((LANES,), jnp.float32)
                for e in range(LANES):
                    q = g * LANES + e
                    pa, pb = plsc.unpack(
                        srow[r, q, :] * drow[r, q, :],
                        format=plsc.PackFormat.INTERLEAVED,
                        preferred_element_type=jnp.float32)
                    acc = jnp.where(lane == e, jnp.sum(pa + pb), acc)
                obuf[r, pl.ds(g * LANES, LANES)] = 1.0 / (1.0 + jnp.exp(-acc))
                return c2
            lax.fori_loop(0, n_groups, group_body, 0, unroll=False)

        def fire_out(b, r):
            for j in range(GPB):
                blk = base_blk + b * GPB + j
                pltpu.async_copy(obuf.at[r, pl.ds(j * GROUP, GROUP)],
                                 out_hbm.at[pl.ds(blk * GROUP, GROUP)],
                                 osems[r])

        def drain_out(r):
            for j in range(GPB):
                pltpu.make_async_copy(obuf.at[r, pl.ds(j * GROUP, GROUP)],
                                      out_hbm.at[pl.ds(0, GROUP)],
                                      osems[r]).wait()

        def compute(b, r):
            @pl.when(b >= DEPTH)
            def _():
                drain_out(r)
            dot_groups(r, BATCH // LANES)
            fire_out(b, r)

        def one_block(blk, valid):
            @pl.when(valid)
            def _():
                pltpu.async_copy(ei_hbm.at[blk, 0, :], sidx.at[0], isems[0])
                pltpu.async_copy(ei_hbm.at[blk, 1, :], didx.at[0], isems[0])
                pltpu.make_async_copy(ei_hbm.at[0, 0, :], sidx.at[0],
                                      isems[0]).wait()
                pltpu.make_async_copy(ei_hbm.at[0, 1, :], didx.at[0],
                                      isems[0]).wait()
                pltpu.async_copy(z_hbm.at[sidx.at[0]],
                                 srow.at[0, pl.ds(0, GROUP), :], gsems[0])
                pltpu.async_copy(z_hbm.at[didx.at[0]],
                                 drow.at[0, pl.ds(0, GROUP), :], gsems[0])
                pltpu.make_async_copy(z_hbm.at[sidx.at[0]],
                                      srow.at[0, pl.ds(0, GROUP), :],
                                      gsems[0]).wait()
                pltpu.make_async_copy(z_hbm.at[didx.at[0]],
                                      drow.at[0, pl.ds(0, GROUP), :],
                                      gsems[0]).wait()
                dot_groups(0, GROUP // LANES)
                pltpu.async_copy(obuf.at[0, pl.ds(0, GROUP)],
                                 out_hbm.at[pl.ds(blk * GROUP, GROUP)],
                                 osems[0])
                pltpu.make_async_copy(obuf.at[0, pl.ds(0, GROUP)],
                                      out_hbm.at[pl.ds(0, GROUP)],
                                      osems[0]).wait()

        for t in range(tail_lo):
            one_block(base_blk + n_batches * GPB + t, True)
        one_block(base_blk + n_batches * GPB + tail_lo, n_my_extra > 0)

        fire_idx(0, 0)
        fire_idx(1, 1)
        fire_idx(2, 2)
        drain_idx(0)
        fire_gathers(0)
        drain_idx(1)
        fire_gathers(1)

        def stepper(b, r, do_g, do_i):
            if do_g:
                drain_idx((r + 2) % DEPTH)
                fire_gathers((r + 2) % DEPTH)
            drain_gathers(r)
            if do_i:
                fire_idx(b + 3, r)
            compute(b, r)

        m3 = ((n_batches - 3) // 3) * 3

        def loop_body(i, carry):
            b0 = i * 3
            stepper(b0, 0, True, True)
            stepper(b0 + 1, 1, True, True)
            stepper(b0 + 2, 2, True, True)
            return carry

        lax.fori_loop(0, m3 // 3, loop_body, 0, unroll=False)
        for b in range(m3, n_batches):
            stepper(b, b % 3, b + 2 <= n_batches - 1, b + 3 <= n_batches - 1)
        for r in range(DEPTH):
            drain_out(r)

    return k


def kernel(z, edge_index):
    n_nodes, d = z.shape
    e = edge_index.shape[1]
    nblk = e // GROUP
    ei = edge_index.astype(jnp.int32)
    ei3 = ei.reshape(2, nblk, GROUP).transpose(1, 0, 2)
    zb = z.reshape(-1).astype(jnp.bfloat16).reshape(n_nodes, d)
    return _make_sc_kernel(n_nodes, d, e)(zb, ei3)
